# Initial kernel scaffold; baseline (speedup 1.0000x reference)
import jax
import jax.numpy as jnp
from jax.experimental import pallas as pl


def kernel(x, edge_index, W, b, gamma, beta):
    raise NotImplementedError("write your pallas kernel here")


# speedup vs baseline: 23.0742x; 23.0742x over previous
import functools

import jax
import jax.numpy as jnp
from jax import lax
from jax.experimental import pallas as pl
from jax.experimental.pallas import tpu as pltpu
from jax.experimental.pallas import tpu_sc as plsc

NC = 2
NS = 16
LANES = 16

CHUNK = 128


def _sc_mesh():
    return plsc.VectorSubcoreMesh(
        core_axis_name="c", subcore_axis_name="s", num_cores=NC, num_subcores=NS
    )


def _degree_kernel(n_nodes, n_edges):
    edges_per_tile = n_edges // (NC * NS)
    n_full = edges_per_tile // CHUNK
    tail = edges_per_tile - n_full * CHUNK

    @functools.partial(
        pl.kernel,
        out_type=jax.ShapeDtypeStruct((NC * n_nodes,), jnp.float32),
        mesh=_sc_mesh(),
        scratch_types=[
            pltpu.VMEM((CHUNK,), jnp.int32),
            pltpu.VMEM((CHUNK,), jnp.float32),
            pltpu.VMEM((16,), jnp.int32),
            pltpu.VMEM((1024,), jnp.float32),
            pltpu.VMEM_SHARED((n_nodes,), jnp.float32),
        ],
    )
    def deg_kernel(dst_hbm, out_hbm, idx_v, ones_v, idx_t, zbuf, acc_sh):
        c = lax.axis_index("c")
        s = lax.axis_index("s")
        tile = c * NS + s
        base = tile * edges_per_tile

        def fill_ones(i, _):
            ones_v[pl.ds(i * LANES, LANES)] = jnp.ones((LANES,), jnp.float32)
            return 0
        lax.fori_loop(0, CHUNK // LANES, fill_ones, 0)

        def fill_zero(i, _):
            zbuf[pl.ds(i * LANES, LANES)] = jnp.zeros((LANES,), jnp.float32)
            return 0
        lax.fori_loop(0, 1024 // LANES, fill_zero, 0)

        @pl.when(s == 0)
        def _():
            n_z = n_nodes // 1024

            def zero_acc(i, _):
                pltpu.sync_copy(zbuf, acc_sh.at[pl.ds(i * 1024, 1024)])
                return 0
            lax.fori_loop(0, n_z, zero_acc, 0)
            rem = n_nodes - n_z * 1024
            if rem:
                pltpu.sync_copy(
                    zbuf.at[pl.ds(0, rem)], acc_sh.at[pl.ds(n_z * 1024, rem)]
                )

        plsc.subcore_barrier()

        def body(j, _):
            off = base + j * CHUNK
            pltpu.sync_copy(dst_hbm.at[pl.ds(off, CHUNK)], idx_v)
            pltpu.sync_copy(ones_v, acc_sh.at[idx_v], add=True)
            return 0
        lax.fori_loop(0, n_full, body, 0)

        if tail:
            pltpu.sync_copy(
                dst_hbm.at[pl.ds(base + n_full * CHUNK, tail)], idx_t
            )
            pltpu.sync_copy(ones_v.at[pl.ds(0, tail)], acc_sh.at[idx_t], add=True)

        plsc.subcore_barrier()

        n_oc = n_nodes // 1024
        oc_per_tile = (n_oc + NS - 1) // NS

        def copy_out(i, _):
            k = s + i * NS

            @pl.when(k < n_oc)
            def _():
                pltpu.sync_copy(acc_sh.at[pl.ds(k * 1024, 1024)], zbuf)
                pltpu.sync_copy(
                    zbuf, out_hbm.at[pl.ds(c * n_nodes + k * 1024, 1024)]
                )
            return 0
        lax.fori_loop(0, oc_per_tile, copy_out, 0)
        rem = n_nodes - n_oc * 1024
        if rem:
            @pl.when(s == NS - 1)
            def _():
                pltpu.sync_copy(acc_sh.at[pl.ds(n_oc * 1024, rem)], zbuf.at[pl.ds(0, rem)])
                pltpu.sync_copy(
                    zbuf.at[pl.ds(0, rem)],
                    out_hbm.at[pl.ds(c * n_nodes + n_oc * 1024, rem)],
                )

    return deg_kernel


def _scatter_kernel(n_nodes, n_edges, d):
    edges_per_tile = n_edges // (NC * NS)
    n_full = edges_per_tile // CHUNK
    tail = edges_per_tile - n_full * CHUNK
    n_row_chunks = n_nodes // CHUNK
    row_tail = n_nodes - n_row_chunks * CHUNK
    chunks_per_tile = (n_row_chunks + NS - 1) // NS

    @functools.partial(
        pl.kernel,
        out_type=jax.ShapeDtypeStruct((NC * n_nodes, d), jnp.float32),
        mesh=_sc_mesh(),
        scratch_types=[
            pltpu.VMEM((CHUNK,), jnp.int32),
            pltpu.VMEM((CHUNK,), jnp.int32),
            pltpu.VMEM((CHUNK, d), jnp.float32),
            pltpu.VMEM((16,), jnp.int32),
            pltpu.VMEM((16,), jnp.int32),
            pltpu.VMEM_SHARED((n_nodes, d), jnp.float32),
            pltpu.SemaphoreType.DMA,
        ],
    )
    def scat_kernel(src_hbm, dst_hbm, g_hbm, out_hbm,
                    sidx, didx, rows, sidx_t, didx_t, acc_sh, sem):
        c = lax.axis_index("c")
        s = lax.axis_index("s")
        tile = c * NS + s
        base = tile * edges_per_tile

        def fill_row(i, _):
            def fill_lane(j, _):
                rows[i, pl.ds(j * LANES, LANES)] = jnp.zeros((LANES,), jnp.float32)
                return 0
            lax.fori_loop(0, d // LANES, fill_lane, 0)
            return 0
        lax.fori_loop(0, CHUNK, fill_row, 0)

        def zero_rows(i, _):
            k = s + i * NS

            @pl.when(k < n_row_chunks)
            def _():
                pltpu.sync_copy(rows, acc_sh.at[pl.ds(k * CHUNK, CHUNK)])
            return 0
        lax.fori_loop(0, chunks_per_tile, zero_rows, 0)
        if row_tail:
            @pl.when(s == 0)
            def _():
                pltpu.sync_copy(
                    rows.at[pl.ds(0, row_tail)],
                    acc_sh.at[pl.ds(n_row_chunks * CHUNK, row_tail)],
                )

        plsc.subcore_barrier()

        def body(j, _):
            off = base + j * CHUNK
            pltpu.sync_copy(src_hbm.at[pl.ds(off, CHUNK)], sidx)
            pltpu.sync_copy(dst_hbm.at[pl.ds(off, CHUNK)], didx)
            pltpu.async_copy(g_hbm.at[sidx], rows, sem).wait()
            pltpu.sync_copy(rows, acc_sh.at[didx], add=True)
            return 0
        lax.fori_loop(0, n_full, body, 0)

        if tail:
            off = base + n_full * CHUNK
            pltpu.sync_copy(src_hbm.at[pl.ds(off, tail)], sidx_t)
            pltpu.sync_copy(dst_hbm.at[pl.ds(off, tail)], didx_t)
            pltpu.async_copy(
                g_hbm.at[sidx_t], rows.at[pl.ds(0, tail)], sem
            ).wait()
            pltpu.sync_copy(rows.at[pl.ds(0, tail)], acc_sh.at[didx_t], add=True)

        plsc.subcore_barrier()

        out_base = c * n_nodes

        def copy_out(i, _):
            k = s + i * NS

            @pl.when(k < n_row_chunks)
            def _():
                pltpu.sync_copy(
                    acc_sh.at[pl.ds(k * CHUNK, CHUNK)],
                    out_hbm.at[pl.ds(out_base + k * CHUNK, CHUNK)],
                )
            return 0
        lax.fori_loop(0, chunks_per_tile, copy_out, 0)
        if row_tail:
            @pl.when(s == 0)
            def _():
                pltpu.sync_copy(
                    acc_sh.at[pl.ds(n_row_chunks * CHUNK, row_tail)],
                    out_hbm.at[pl.ds(out_base + n_row_chunks * CHUNK, row_tail)],
                )

    return scat_kernel


def _gW_body(x_ref, w_ref, degp_ref, g_ref):
    n = x_ref.shape[0]
    h = jnp.dot(x_ref[...], w_ref[...], preferred_element_type=jnp.float32)
    deg = degp_ref[0:n] + degp_ref[n:2 * n] + 1.0
    dis = lax.rsqrt(deg)
    g_ref[...] = h * dis[:, None]


def _epilogue_body(s_ref, g_ref, degp_ref, b_ref, gamma_ref, beta_ref, y_ref):
    n = g_ref.shape[0]
    deg = degp_ref[0:n] + degp_ref[n:2 * n] + 1.0
    dis = lax.rsqrt(deg)
    total = s_ref[0:n, :] + s_ref[n:2 * n, :] + g_ref[...]
    out = total * dis[:, None] + b_ref[...][None, :]
    mean = jnp.mean(out, axis=0)
    var = jnp.mean((out - mean[None, :]) ** 2, axis=0)
    y = gamma_ref[...][None, :] * (out - mean[None, :]) * lax.rsqrt(
        var[None, :] + 1e-5
    ) + beta_ref[...][None, :]
    y_ref[...] = jnp.maximum(y, 0.0)


def kernel(x, edge_index, W, b, gamma, beta):
    n_nodes, d_in = x.shape
    d_out = W.shape[1]
    n_edges = edge_index.shape[1]
    src = edge_index[0]
    dst = edge_index[1]

    degp = _degree_kernel(n_nodes, n_edges)(dst)

    g = pl.pallas_call(
        _gW_body,
        out_shape=jax.ShapeDtypeStruct((n_nodes, d_out), jnp.float32),
    )(x, W, degp)

    s_partial = _scatter_kernel(n_nodes, n_edges, d_out)(src, dst, g)

    y = pl.pallas_call(
        _epilogue_body,
        out_shape=jax.ShapeDtypeStruct((n_nodes, d_out), jnp.float32),
    )(s_partial, g, degp, b, gamma, beta)
    return y
